# baseline (speedup 1.0000x reference)
import functools

import jax
import jax.numpy as jnp
from jax import lax
from jax.experimental import pallas as pl
from jax.experimental.pallas import tpu as pltpu
from jax.experimental.pallas import tpu_sc as plsc

_HIDDEN = 1024
_NC = 2
_NS = 16
_NW = _NC * _NS
_CHUNK = 32
_NBUF = 3
_L = 16


@functools.cache
def _sc_lookup(total: int, n_rows: int):
    per_w = total // _NW
    nch = per_w // _CHUNK
    nblk = _HIDDEN // _L
    mesh = plsc.VectorSubcoreMesh(core_axis_name="c", subcore_axis_name="s")

    @functools.partial(
        pl.kernel,
        out_type=jax.ShapeDtypeStruct((total, _HIDDEN), jnp.float32),
        mesh=mesh,
        compiler_params=pltpu.CompilerParams(
            use_tc_tiling_on_sc=False, needs_layout_passes=False
        ),
        scratch_types=[
            pltpu.VMEM((per_w,), jnp.int32),
            pltpu.VMEM((n_rows, _HIDDEN), jnp.float32),
            *[pltpu.VMEM((_CHUNK, _HIDDEN), jnp.float32) for _ in range(_NBUF)],
            pltpu.SemaphoreType.DMA,
            *[pltpu.SemaphoreType.DMA for _ in range(_NBUF)],
        ],
    )
    def k(ids_hbm, table_hbm, out_hbm, idx_v, table_v, *rest):
        bufs = rest[:_NBUF]
        gsem = rest[_NBUF]
        ssems = rest[_NBUF + 1:]
        wid = lax.axis_index("s") * _NC + lax.axis_index("c")
        base = wid * per_w
        cp_t = pltpu.async_copy(table_hbm, table_v, gsem)
        pltpu.sync_copy(ids_hbm.at[wid], idx_v)
        cp_t.wait()
        iota = lax.iota(jnp.int32, _L)
        scp = [None] * nch
        for c in range(nch):
            slot = c % _NBUF
            if c >= _NBUF:
                scp[c - _NBUF].wait()
            buf = bufs[slot]

            def row_body(r, _, buf=buf, c=c):
                vec = idx_v[pl.ds(c * _CHUNK + (r // _L) * _L, _L)]
                rid = jnp.max(jnp.where(iota == r % _L, vec, 0))

                @plsc.parallel_loop(0, nblk, 1, unroll=8)
                def blk_body(t, buf=buf, r=r, rid=rid):
                    off = t * _L
                    buf[r, pl.ds(off, _L)] = table_v[rid, pl.ds(off, _L)]
                return ()

            lax.fori_loop(0, _CHUNK, row_body, (), unroll=False)
            scp[c] = pltpu.async_copy(
                buf,
                out_hbm.at[pl.ds(base + c * _CHUNK, _CHUNK)],
                ssems[slot],
            )
        for c in range(nch - _NBUF, nch):
            scp[c].wait()

    return k


def kernel(ids, indicator_embs):
    b, s = ids.shape
    total = b * s
    ids_w = ids.astype(jnp.int32).reshape(_NW, total // _NW)
    out = _sc_lookup(total, indicator_embs.shape[0])(ids_w, indicator_embs)
    return out.reshape(b, s, _HIDDEN)


# speedup vs baseline: 1.1079x; 1.1079x over previous
import functools

import jax
import jax.numpy as jnp
from jax import lax
from jax.experimental import pallas as pl
from jax.experimental.pallas import tpu as pltpu
from jax.experimental.pallas import tpu_sc as plsc

_HIDDEN = 1024
_NC = 2
_NS = 16
_NW = _NC * _NS
_L = 16


@functools.cache
def _sc_lookup(total: int, n_rows: int):
    per_w = total // _NW
    mesh = plsc.VectorSubcoreMesh(core_axis_name="c", subcore_axis_name="s")

    @functools.partial(
        pl.kernel,
        out_type=jax.ShapeDtypeStruct((total, _HIDDEN), jnp.float32),
        mesh=mesh,
        compiler_params=pltpu.CompilerParams(
            use_tc_tiling_on_sc=False, needs_layout_passes=False
        ),
        scratch_types=[
            pltpu.VMEM((per_w,), jnp.int32),
            pltpu.VMEM((n_rows, _HIDDEN), jnp.float32),
            pltpu.SemaphoreType.DMA,
            pltpu.SemaphoreType.DMA,
        ],
    )
    def k(ids_hbm, table_hbm, out_hbm, idx_v, table_v, tsem, rsem):
        wid = lax.axis_index("s") * _NC + lax.axis_index("c")
        base = wid * per_w
        cp_t = pltpu.async_copy(table_hbm, table_v, tsem)
        pltpu.sync_copy(ids_hbm.at[wid], idx_v)
        cp_t.wait()
        iota = lax.iota(jnp.int32, _L)

        def fire(r, _):
            vec = idx_v[pl.ds((r // _L) * _L, _L)]
            rid = jnp.max(jnp.where(iota == r % _L, vec, 0))
            pltpu.async_copy(table_v.at[rid], out_hbm.at[base + r], rsem)
            return ()

        lax.fori_loop(0, per_w, fire, (), unroll=False)

        def drain(r, _):
            pltpu.make_async_copy(table_v.at[0], out_hbm.at[base], rsem).wait()
            return ()

        lax.fori_loop(0, per_w, drain, (), unroll=False)

    return k


def kernel(ids, indicator_embs):
    b, s = ids.shape
    total = b * s
    ids_w = ids.astype(jnp.int32).reshape(_NW, total // _NW)
    out = _sc_lookup(total, indicator_embs.shape[0])(ids_w, indicator_embs)
    return out.reshape(b, s, _HIDDEN)
